# baseline (speedup 1.0000x reference)
import functools

import jax
import jax.numpy as jnp
from jax import lax
from jax.experimental import pallas as pl
from jax.experimental.pallas import tpu as pltpu
from jax.experimental.pallas import tpu_sc as plsc

S = 256
S3 = S * S * S
K3 = 27
KO = 28
NC, NS = 2, 16
NW = NC * NS
LANES = 16


def _sc_mesh():
    return plsc.VectorSubcoreMesh(core_axis_name="c", subcore_axis_name="s")


_SC_PARAMS = pltpu.CompilerParams(
    use_tc_tiling_on_sc=False, needs_layout_passes=False)


def _worker_id():
    return lax.axis_index("s") * NC + lax.axis_index("c")


def _make_scatter_table(CH, TS):
    @functools.partial(
        pl.kernel,
        out_type=jax.ShapeDtypeStruct((TS,), jnp.int32),
        mesh=_sc_mesh(),
        compiler_params=_SC_PARAMS,
        scratch_types=[
            pltpu.VMEM((CH,), jnp.int32),
            pltpu.VMEM((CH,), jnp.int32),
            pltpu.SemaphoreType.DMA,
        ],
    )
    def scatter_table(keys1d, rows1d, table, idx_v, val_v, sem):
        base = _worker_id() * CH
        pltpu.sync_copy(keys1d.at[pl.ds(base, CH)], idx_v)
        pltpu.sync_copy(rows1d.at[pl.ds(base, CH)], val_v)
        pltpu.make_async_copy(val_v, table.at[idx_v], sem).start()
        pltpu.make_async_copy(val_v, table.at[idx_v], sem).wait()

    return scatter_table


def _make_build_map(CH, Np, N, TS):
    VN = CH // LANES

    @functools.partial(
        pl.kernel,
        out_type=jax.ShapeDtypeStruct((KO * Np,), jnp.int32),
        mesh=_sc_mesh(),
        compiler_params=_SC_PARAMS,
        scratch_types=[
            pltpu.VMEM((Np,), jnp.int32),
            pltpu.VMEM((CH,), jnp.int32),
            pltpu.VMEM((CH,), jnp.int32),
            pltpu.VMEM((CH,), jnp.int32),
            pltpu.VMEM((CH,), jnp.int32),
            pltpu.VMEM((CH,), jnp.int32),
            pltpu.SemaphoreType.DMA,
        ],
    )
    def build_map(table, keys1d, vmask1d, src_all,
                  keysf, vm_v, qidx, qfull, cand, srcb, sem):
        base = _worker_id() * CH
        pltpu.sync_copy(keys1d, keysf)
        pltpu.sync_copy(vmask1d.at[pl.ds(base, CH)], vm_v)

        def per_k(k, carry):
            dx = k // 9 - 1
            dy = (k // 3) % 3 - 1
            dz = k % 3 - 1
            dkey = (dx * S + dy) * S + dz

            def pass1(v, c):
                sl = pl.ds(v * LANES, LANES)
                key = keysf[pl.ds(base + v * LANES, LANES)]
                vm = vm_v[sl]
                q = key + dkey
                valid = ((vm >> k) & 1) > 0
                qidx[sl] = jnp.where(valid, q, 0)
                qfull[sl] = jnp.where(valid, q, -1)
                return c

            lax.fori_loop(0, VN, pass1, 0)

            pltpu.make_async_copy(table.at[qidx], cand, sem).start()
            pltpu.make_async_copy(table.at[qidx], cand, sem).wait()

            def pass2(v, c):
                sl = pl.ds(v * LANES, LANES)
                cd = cand[sl]
                cdc = jnp.clip(cd, 0, Np - 1)
                kv = plsc.load_gather(keysf, [cdc])
                qv = qfull[sl]
                srcb[sl] = jnp.where(kv == qv, cdc, N)
                return c

            lax.fori_loop(0, VN, pass2, 0)
            pltpu.sync_copy(srcb, src_all.at[pl.ds(k * Np + base, CH)])
            return carry

        lax.fori_loop(0, KO, per_k, 0)

    return build_map


def _make_gather_rows(Np, Ch):
    CH2 = Np // NS
    NpP = Np // 8
    CHP = CH2 // 8

    @functools.partial(
        pl.kernel,
        out_type=jax.ShapeDtypeStruct((2, KO, Np, Ch), jnp.bfloat16),
        mesh=_sc_mesh(),
        compiler_params=_SC_PARAMS,
        scratch_types=[
            pltpu.VMEM_SHARED((Np, Ch), jnp.bfloat16),
            pltpu.VMEM((CH2,), jnp.int32),
            pltpu.VMEM((CH2, Ch), jnp.bfloat16),
            pltpu.SemaphoreType.DMA,
        ],
    )
    def gather_rows(src_all, fxs, g_out, spm, idx_v, rows_v, sem):
        cid = lax.axis_index("c")
        sid = lax.axis_index("s")
        rowbase = sid * CH2
        pltpu.sync_copy(fxs.at[cid, pl.ds(rowbase, CH2)],
                        spm.at[pl.ds(rowbase, CH2)])
        plsc.subcore_barrier()

        def per_k(k, carry):
            pltpu.sync_copy(src_all.at[pl.ds(k * Np + rowbase, CH2)], idx_v)
            pltpu.make_async_copy(spm.at[idx_v], rows_v, sem).start()
            pltpu.make_async_copy(spm.at[idx_v], rows_v, sem).wait()
            pltpu.sync_copy(rows_v, g_out.at[cid, k, pl.ds(rowbase, CH2)])
            return carry

        lax.fori_loop(0, KO, per_k, 0)

    return gather_rows


def _matmul(g, wblk, NpP, bmr=256):
    def mm(g_ref, w_ref, o_ref):
        acc = jnp.zeros((bmr, 256), jnp.float32)
        for c in range(2):
            for k in range(KO):
                acc = acc + jnp.dot(g_ref[c, k], w_ref[c * KO + k],
                                    preferred_element_type=jnp.float32)
        acch = acc.astype(jnp.bfloat16)
        o_ref[0] = acch[:, :128]
        o_ref[1] = acch[:, 128:]

    return pl.pallas_call(
        mm,
        grid=(NpP // bmr,),
        in_specs=[
            pl.BlockSpec((2, KO, bmr, 128), lambda m: (0, 0, m, 0)),
            pl.BlockSpec((2 * KO, 128, 256), lambda m: (0, 0, 0)),
        ],
        out_specs=pl.BlockSpec((2, bmr, 128), lambda m: (0, m, 0)),
        out_shape=jax.ShapeDtypeStruct((2, NpP, 128), jnp.bfloat16),
    )(g, wblk)


def kernel(feats, coords, W1, W2):
    N, C = feats.shape
    Ch = C // 2
    CH = -(-N // (NW * 128)) * 128
    Np = NW * CH
    NpP = Np // 8
    pad = Np - N
    TS = S3 + 2432

    x = coords[:, 0].astype(jnp.int32)
    y = coords[:, 1].astype(jnp.int32)
    z = coords[:, 2].astype(jnp.int32)
    keys = (x * S + y) * S + z
    keys_p = jnp.concatenate([keys, S3 + jnp.arange(pad, dtype=jnp.int32)])
    row_ids = jnp.arange(Np, dtype=jnp.int32)

    vmask = jnp.zeros((N,), dtype=jnp.int32)
    kk = 0
    vx = {-1: x > 0, 0: jnp.ones_like(x, dtype=bool), 1: x < S - 1}
    vy = {-1: y > 0, 0: jnp.ones_like(y, dtype=bool), 1: y < S - 1}
    vz = {-1: z > 0, 0: jnp.ones_like(z, dtype=bool), 1: z < S - 1}
    for dx in (-1, 0, 1):
        for dy in (-1, 0, 1):
            for dz in (-1, 0, 1):
                vmask = vmask | ((vx[dx] & vy[dy] & vz[dz]).astype(jnp.int32) << kk)
                kk += 1
    vmask_p = jnp.concatenate([vmask, jnp.zeros((pad,), dtype=jnp.int32)])

    fx0 = jnp.concatenate([feats, jnp.zeros((pad, C), dtype=feats.dtype)])
    fxs1 = fx0.reshape(Np, 2, Ch).transpose(1, 0, 2).astype(jnp.bfloat16)

    def wblk(W):
        Wp = jnp.concatenate([W, jnp.zeros((KO - K3, C, C), dtype=W.dtype)])
        A = (Wp.reshape(KO, 2, Ch, C).transpose(1, 0, 2, 3)
             .reshape(2 * KO, Ch, 2, Ch))
        eye8 = jnp.eye(8, dtype=W.dtype)
        blk = (eye8[None, :, None, None, :, None]
               * A[:, None, :, :, None, :])
        return blk.reshape(2 * KO, 128, 256).astype(jnp.bfloat16)

    table = _make_scatter_table(CH, TS)(keys_p, row_ids)
    src_all = _make_build_map(CH, Np, N, TS)(table, keys_p, vmask_p)

    gather = _make_gather_rows(Np, Ch)

    def layer(fxs, W):
        g = gather(src_all, fxs)
        gp = g.reshape(2, KO, NpP, 128)
        hp = _matmul(gp, wblk(W), NpP)
        return hp.reshape(2, Np, Ch)

    h1s = layer(fxs1, W1)
    h2s = layer(h1s, W2)
    h2 = (h2s.transpose(1, 0, 2).reshape(Np, C)[:N].astype(jnp.float32))
    return h2


# speedup vs baseline: 5.7866x; 5.7866x over previous
import functools

import jax
import jax.numpy as jnp
from jax import lax
from jax.experimental import pallas as pl
from jax.experimental.pallas import tpu as pltpu
from jax.experimental.pallas import tpu_sc as plsc

S = 256
S3 = S * S * S
K3 = 27
KO = 28
NC, NS = 2, 16
NW = NC * NS
LANES = 16


def _sc_mesh():
    return plsc.VectorSubcoreMesh(core_axis_name="c", subcore_axis_name="s")


_SC_PARAMS = pltpu.CompilerParams(
    use_tc_tiling_on_sc=False, needs_layout_passes=False)


def _worker_id():
    return lax.axis_index("s") * NC + lax.axis_index("c")


def _make_scatter_table(CH, TS):
    @functools.partial(
        pl.kernel,
        out_type=jax.ShapeDtypeStruct((TS,), jnp.int32),
        mesh=_sc_mesh(),
        compiler_params=_SC_PARAMS,
        scratch_types=[
            pltpu.VMEM((CH,), jnp.int32),
            pltpu.VMEM((CH,), jnp.int32),
            pltpu.SemaphoreType.DMA,
        ],
    )
    def scatter_table(keys1d, rows1d, table, idx_v, val_v, sem):
        base = _worker_id() * CH
        pltpu.sync_copy(keys1d.at[pl.ds(base, CH)], idx_v)
        pltpu.sync_copy(rows1d.at[pl.ds(base, CH)], val_v)
        pltpu.make_async_copy(val_v, table.at[idx_v], sem).start()
        pltpu.make_async_copy(val_v, table.at[idx_v], sem).wait()

    return scatter_table


def _make_build_map(CH, Np, N, TS):
    VN = CH // LANES

    @functools.partial(
        pl.kernel,
        out_type=jax.ShapeDtypeStruct((KO * Np,), jnp.int32),
        mesh=_sc_mesh(),
        compiler_params=_SC_PARAMS,
        scratch_types=[
            pltpu.VMEM((Np,), jnp.int32),
            pltpu.VMEM((CH,), jnp.int32),
            pltpu.VMEM((CH,), jnp.int32),
            pltpu.VMEM((CH,), jnp.int32),
            pltpu.VMEM((CH,), jnp.int32),
            pltpu.VMEM((CH,), jnp.int32),
            pltpu.SemaphoreType.DMA,
        ],
    )
    def build_map(table, keys1d, vmask1d, src_all,
                  keysf, vm_v, qidx, qfull, cand, srcb, sem):
        base = _worker_id() * CH
        pltpu.sync_copy(keys1d, keysf)
        pltpu.sync_copy(vmask1d.at[pl.ds(base, CH)], vm_v)

        def per_k(k, carry):
            dx = k // 9 - 1
            dy = (k // 3) % 3 - 1
            dz = k % 3 - 1
            dkey = (dx * S + dy) * S + dz

            def pass1(v, c):
                sl = pl.ds(v * LANES, LANES)
                key = keysf[pl.ds(base + v * LANES, LANES)]
                vm = vm_v[sl]
                q = key + dkey
                valid = ((vm >> k) & 1) > 0
                qidx[sl] = jnp.where(valid, q, 0)
                qfull[sl] = jnp.where(valid, q, -1)
                return c

            lax.fori_loop(0, VN, pass1, 0)

            pltpu.make_async_copy(table.at[qidx], cand, sem).start()
            pltpu.make_async_copy(table.at[qidx], cand, sem).wait()

            def pass2(v, c):
                sl = pl.ds(v * LANES, LANES)
                cd = cand[sl]
                cdc = jnp.clip(cd, 0, Np - 1)
                kv = plsc.load_gather(keysf, [cdc])
                qv = qfull[sl]
                srcb[sl] = jnp.where(kv == qv, cdc, N)
                return c

            lax.fori_loop(0, VN, pass2, 0)
            pltpu.sync_copy(srcb, src_all.at[pl.ds(k * Np + base, CH)])
            return carry

        lax.fori_loop(0, KO, per_k, 0)

    return build_map


def _make_gather_rows(Np, Ch):
    CH2 = Np // NS
    NpP = Np // 8
    CHP = CH2 // 8

    @functools.partial(
        pl.kernel,
        out_type=jax.ShapeDtypeStruct((2, KO, Np, Ch), jnp.bfloat16),
        mesh=_sc_mesh(),
        compiler_params=_SC_PARAMS,
        scratch_types=[
            pltpu.VMEM_SHARED((Np, Ch), jnp.bfloat16),
            pltpu.VMEM((CH2,), jnp.int32),
            pltpu.VMEM((CH2, Ch), jnp.bfloat16),
            pltpu.SemaphoreType.DMA,
        ],
    )
    def gather_rows(src_all, fxs, g_out, spm, idx_v, rows_v, sem):
        cid = lax.axis_index("c")
        sid = lax.axis_index("s")
        rowbase = sid * CH2
        pltpu.sync_copy(fxs.at[cid, pl.ds(rowbase, CH2)],
                        spm.at[pl.ds(rowbase, CH2)])
        plsc.subcore_barrier()

        def per_k(k, carry):
            pltpu.sync_copy(src_all.at[pl.ds(k * Np + rowbase, CH2)], idx_v)
            pltpu.make_async_copy(spm.at[idx_v], rows_v, sem).start()
            pltpu.make_async_copy(spm.at[idx_v], rows_v, sem).wait()
            pltpu.sync_copy(rows_v, g_out.at[cid, k, pl.ds(rowbase, CH2)])
            return carry

        lax.fori_loop(0, KO, per_k, 0)

    return gather_rows


def _matmul(g, wsp, Np, C, bm=2048):
    Ch = C // 2

    def mm(g_ref, w_ref, o_ref):
        acc = jnp.zeros((bm, C), jnp.float32)
        for c in range(2):
            for k in range(KO):
                acc = acc + jnp.dot(g_ref[c, k], w_ref[c * KO + k],
                                    preferred_element_type=jnp.float32)
        acch = acc.astype(jnp.bfloat16)
        o_ref[0] = acch[:, :Ch]
        o_ref[1] = acch[:, Ch:]

    return pl.pallas_call(
        mm,
        grid=(Np // bm,),
        in_specs=[
            pl.BlockSpec((2, KO, bm, Ch), lambda m: (0, 0, m, 0)),
            pl.BlockSpec((2 * KO, Ch, C), lambda m: (0, 0, 0)),
        ],
        out_specs=pl.BlockSpec((2, bm, Ch), lambda m: (0, m, 0)),
        out_shape=jax.ShapeDtypeStruct((2, Np, Ch), jnp.bfloat16),
    )(g, wsp)


def kernel(feats, coords, W1, W2):
    N, C = feats.shape
    Ch = C // 2
    CH = -(-N // (NW * 128)) * 128
    Np = NW * CH
    NpP = Np // 8
    pad = Np - N
    TS = S3 + 2432

    x = coords[:, 0].astype(jnp.int32)
    y = coords[:, 1].astype(jnp.int32)
    z = coords[:, 2].astype(jnp.int32)
    keys = (x * S + y) * S + z
    keys_p = jnp.concatenate([keys, S3 + jnp.arange(pad, dtype=jnp.int32)])
    row_ids = jnp.arange(Np, dtype=jnp.int32)

    vmask = jnp.zeros((N,), dtype=jnp.int32)
    kk = 0
    vx = {-1: x > 0, 0: jnp.ones_like(x, dtype=bool), 1: x < S - 1}
    vy = {-1: y > 0, 0: jnp.ones_like(y, dtype=bool), 1: y < S - 1}
    vz = {-1: z > 0, 0: jnp.ones_like(z, dtype=bool), 1: z < S - 1}
    for dx in (-1, 0, 1):
        for dy in (-1, 0, 1):
            for dz in (-1, 0, 1):
                vmask = vmask | ((vx[dx] & vy[dy] & vz[dz]).astype(jnp.int32) << kk)
                kk += 1
    vmask_p = jnp.concatenate([vmask, jnp.zeros((pad,), dtype=jnp.int32)])

    fx0 = jnp.concatenate([feats, jnp.zeros((pad, C), dtype=feats.dtype)])
    fxs1 = fx0.reshape(Np, 2, Ch).transpose(1, 0, 2).astype(jnp.bfloat16)

    def wsplit(W):
        Wp = jnp.concatenate([W, jnp.zeros((KO - K3, C, C), dtype=W.dtype)])
        return (Wp.reshape(KO, 2, Ch, C).transpose(1, 0, 2, 3)
                .reshape(2 * KO, Ch, C).astype(jnp.bfloat16))

    table = _make_scatter_table(CH, TS)(keys_p, row_ids)
    src_all = _make_build_map(CH, Np, N, TS)(table, keys_p, vmask_p)

    gather = _make_gather_rows(Np, Ch)
    g1 = gather(src_all, fxs1)
    h1s = _matmul(g1, wsplit(W1), Np, C)
    g2 = gather(src_all, h1s)
    h2s = _matmul(g2, wsplit(W2), Np, C)
    h2 = (h2s.transpose(1, 0, 2).reshape(Np, C)[:N].astype(jnp.float32))
    return h2


# speedup vs baseline: 6.1189x; 1.0574x over previous
import functools

import jax
import jax.numpy as jnp
from jax import lax
from jax.experimental import pallas as pl
from jax.experimental.pallas import tpu as pltpu
from jax.experimental.pallas import tpu_sc as plsc

S = 256
S3 = S * S * S
K3 = 27
KO = 28
NC, NS = 2, 16
NW = NC * NS
LANES = 16


def _sc_mesh():
    return plsc.VectorSubcoreMesh(core_axis_name="c", subcore_axis_name="s")


_SC_PARAMS = pltpu.CompilerParams(
    use_tc_tiling_on_sc=False, needs_layout_passes=False)


def _worker_id():
    return lax.axis_index("s") * NC + lax.axis_index("c")


def _make_scatter_table(CH, TS):
    @functools.partial(
        pl.kernel,
        out_type=jax.ShapeDtypeStruct((TS,), jnp.int32),
        mesh=_sc_mesh(),
        compiler_params=_SC_PARAMS,
        scratch_types=[
            pltpu.VMEM((CH,), jnp.int32),
            pltpu.VMEM((CH,), jnp.int32),
            pltpu.SemaphoreType.DMA,
        ],
    )
    def scatter_table(keys1d, rows1d, table, idx_v, val_v, sem):
        base = _worker_id() * CH
        pltpu.sync_copy(keys1d.at[pl.ds(base, CH)], idx_v)
        pltpu.sync_copy(rows1d.at[pl.ds(base, CH)], val_v)
        pltpu.make_async_copy(val_v, table.at[idx_v], sem).start()
        pltpu.make_async_copy(val_v, table.at[idx_v], sem).wait()

    return scatter_table


def _make_build_map(CH, Np, N, TS):
    VN = CH // LANES

    @functools.partial(
        pl.kernel,
        out_type=jax.ShapeDtypeStruct((KO * Np,), jnp.int32),
        mesh=_sc_mesh(),
        compiler_params=_SC_PARAMS,
        scratch_types=[
            pltpu.VMEM((Np,), jnp.int32),
            pltpu.VMEM((CH,), jnp.int32),
            pltpu.VMEM((CH,), jnp.int32),
            pltpu.VMEM((CH,), jnp.int32),
            pltpu.VMEM((CH,), jnp.int32),
            pltpu.VMEM((CH,), jnp.int32),
            pltpu.SemaphoreType.DMA,
        ],
    )
    def build_map(table, keys1d, vmask1d, src_all,
                  keysf, vm_v, qidx, qfull, cand, srcb, sem):
        base = _worker_id() * CH
        pltpu.sync_copy(keys1d, keysf)
        pltpu.sync_copy(vmask1d.at[pl.ds(base, CH)], vm_v)

        def per_k(k, carry):
            dx = k // 9 - 1
            dy = (k // 3) % 3 - 1
            dz = k % 3 - 1
            dkey = (dx * S + dy) * S + dz

            def pass1(v, c):
                sl = pl.ds(v * LANES, LANES)
                key = keysf[pl.ds(base + v * LANES, LANES)]
                vm = vm_v[sl]
                q = key + dkey
                valid = ((vm >> k) & 1) > 0
                qidx[sl] = jnp.where(valid, q, 0)
                qfull[sl] = jnp.where(valid, q, -1)
                return c

            lax.fori_loop(0, VN, pass1, 0)

            pltpu.make_async_copy(table.at[qidx], cand, sem).start()
            pltpu.make_async_copy(table.at[qidx], cand, sem).wait()

            def pass2(v, c):
                sl = pl.ds(v * LANES, LANES)
                cd = cand[sl]
                cdc = jnp.clip(cd, 0, Np - 1)
                kv = plsc.load_gather(keysf, [cdc])
                qv = qfull[sl]
                srcb[sl] = jnp.where(kv == qv, cdc, N)
                return c

            lax.fori_loop(0, VN, pass2, 0)
            pltpu.sync_copy(srcb, src_all.at[pl.ds(k * Np + base, CH)])
            return carry

        lax.fori_loop(0, KO, per_k, 0)

    return build_map


def _make_gather_rows(Np, Ch):
    CH2 = Np // NS
    NpP = Np // 8
    CHP = CH2 // 8

    @functools.partial(
        pl.kernel,
        out_type=jax.ShapeDtypeStruct((2, KO, Np, Ch), jnp.bfloat16),
        mesh=_sc_mesh(),
        compiler_params=_SC_PARAMS,
        scratch_types=[
            pltpu.VMEM_SHARED((Np, Ch), jnp.bfloat16),
            pltpu.VMEM((2, CH2 // 2), jnp.int32),
            pltpu.VMEM((2, CH2 // 2, Ch), jnp.bfloat16),
            pltpu.SemaphoreType.DMA,
        ],
    )
    def gather_rows(src_all, fxs, g_out, spm, idx2, rows2, sem):
        cid = lax.axis_index("c")
        sid = lax.axis_index("s")
        rowbase = sid * CH2
        pltpu.sync_copy(fxs.at[cid, pl.ds(rowbase, CH2)],
                        spm.at[pl.ds(rowbase, CH2)])
        plsc.subcore_barrier()

        HC = CH2 // 2
        pltpu.sync_copy(src_all.at[pl.ds(rowbase, HC)], idx2.at[0])
        pltpu.make_async_copy(spm.at[idx2.at[0]], rows2.at[0], sem).start()

        def per_t(t, carry):
            k = t // 2
            h = lax.rem(t, 2)
            b = lax.rem(t, 2)
            bn = lax.rem(t + 1, 2)

            @pl.when(t + 1 < 2 * KO)
            def _prefetch_idx():
                tn = t + 1
                off = (tn // 2) * Np + rowbase + lax.rem(tn, 2) * HC
                pltpu.sync_copy(src_all.at[pl.ds(off, HC)], idx2.at[bn])

            pltpu.make_async_copy(spm.at[idx2.at[b]], rows2.at[b], sem).wait()

            @pl.when(t + 1 < 2 * KO)
            def _next_gather():
                pltpu.make_async_copy(
                    spm.at[idx2.at[bn]], rows2.at[bn], sem).start()

            pltpu.sync_copy(rows2.at[b],
                            g_out.at[cid, k, pl.ds(rowbase + h * HC, HC)])
            return carry

        lax.fori_loop(0, 2 * KO, per_t, 0)

    return gather_rows


def _matmul(g, wsp, Np, C, bm=2048):
    Ch = C // 2

    def mm(g_ref, w_ref, o_ref):
        acc = jnp.zeros((bm, C), jnp.float32)
        for c in range(2):
            for k in range(KO):
                acc = acc + jnp.dot(g_ref[c, k], w_ref[c * KO + k],
                                    preferred_element_type=jnp.float32)
        acch = acc.astype(jnp.bfloat16)
        o_ref[0] = acch[:, :Ch]
        o_ref[1] = acch[:, Ch:]

    return pl.pallas_call(
        mm,
        grid=(Np // bm,),
        in_specs=[
            pl.BlockSpec((2, KO, bm, Ch), lambda m: (0, 0, m, 0)),
            pl.BlockSpec((2 * KO, Ch, C), lambda m: (0, 0, 0)),
        ],
        out_specs=pl.BlockSpec((2, bm, Ch), lambda m: (0, m, 0)),
        out_shape=jax.ShapeDtypeStruct((2, Np, Ch), jnp.bfloat16),
    )(g, wsp)


def kernel(feats, coords, W1, W2):
    N, C = feats.shape
    Ch = C // 2
    CH = -(-N // (NW * LANES)) * LANES
    Np = NW * CH
    NpP = Np // 8
    pad = Np - N
    TS = S3 + 2432

    x = coords[:, 0].astype(jnp.int32)
    y = coords[:, 1].astype(jnp.int32)
    z = coords[:, 2].astype(jnp.int32)
    keys = (x * S + y) * S + z
    keys_p = jnp.concatenate([keys, S3 + jnp.arange(pad, dtype=jnp.int32)])
    row_ids = jnp.arange(Np, dtype=jnp.int32)

    vmask = jnp.zeros((N,), dtype=jnp.int32)
    kk = 0
    vx = {-1: x > 0, 0: jnp.ones_like(x, dtype=bool), 1: x < S - 1}
    vy = {-1: y > 0, 0: jnp.ones_like(y, dtype=bool), 1: y < S - 1}
    vz = {-1: z > 0, 0: jnp.ones_like(z, dtype=bool), 1: z < S - 1}
    for dx in (-1, 0, 1):
        for dy in (-1, 0, 1):
            for dz in (-1, 0, 1):
                vmask = vmask | ((vx[dx] & vy[dy] & vz[dz]).astype(jnp.int32) << kk)
                kk += 1
    vmask_p = jnp.concatenate([vmask, jnp.zeros((pad,), dtype=jnp.int32)])

    fx0 = jnp.concatenate([feats, jnp.zeros((pad, C), dtype=feats.dtype)])
    fxs1 = fx0.reshape(Np, 2, Ch).transpose(1, 0, 2).astype(jnp.bfloat16)

    def wsplit(W):
        Wp = jnp.concatenate([W, jnp.zeros((KO - K3, C, C), dtype=W.dtype)])
        return (Wp.reshape(KO, 2, Ch, C).transpose(1, 0, 2, 3)
                .reshape(2 * KO, Ch, C).astype(jnp.bfloat16))

    table = _make_scatter_table(CH, TS)(keys_p, row_ids)
    src_all = _make_build_map(CH, Np, N, TS)(table, keys_p, vmask_p)

    gather = _make_gather_rows(Np, Ch)
    g1 = gather(src_all, fxs1)
    h1s = _matmul(g1, wsplit(W1), Np, C)
    g2 = gather(src_all, h1s)
    h2s = _matmul(g2, wsplit(W2), Np, C)
    h2 = (h2s.transpose(1, 0, 2).reshape(Np, C)[:N].astype(jnp.float32))
    return h2
